# baseline (speedup 1.0000x reference)
import math

import jax
import jax.numpy as jnp
from jax.experimental import pallas as pl
from jax.experimental.pallas import tpu as pltpu

_V = 100000
_N = 1024
_PAD = 0
_SMOOTH = 0.1
_CONF = 1.0 - _SMOOTH
_FILL = _SMOOTH / (_V - 2)
_C = _FILL * math.log(_FILL) * (_V - 2) + _CONF * math.log(_CONF)

_RB = 32
_NRB = _N // _RB

_GR = 8
_NGB = _N // _GR


def _stream_kernel(x_ref, tgt_ref, out_ref):
    xb = x_ref[...]
    rowsum = jnp.sum(xb, axis=1, keepdims=True)
    x0 = xb[:, 0:1]
    valid = tgt_ref[...] != _PAD
    out_ref[0, 0, 0] = jnp.sum(jnp.where(valid, _C - _FILL * (rowsum - x0), 0.0))


def _make_gather_spec(j):
    return pl.BlockSpec((_GR, 128), lambda i, tgt: (i, tgt[i * _GR + j] // 128))


def _gather_kernel(tgt_sm, *refs):
    i = pl.program_id(0)
    part_ref = refs[_GR]
    out_ref = refs[_GR + 1]
    lane = jax.lax.broadcasted_iota(jnp.int32, (_GR, 128), 1)
    sub = jax.lax.broadcasted_iota(jnp.int32, (_GR, 128), 0)
    s = jnp.float32(0.0)
    for j in range(_GR):
        t = tgt_sm[i * _GR + j]
        sel = (sub == j) & (lane == t % 128)
        v = jnp.sum(jnp.where(sel, refs[j][...], 0.0))
        s += jnp.where(t != _PAD, v, 0.0)

    @pl.when(i == 0)
    def _init():
        out_ref[0, 0] = 0.0

    out_ref[0, 0] += (_FILL - _CONF) * s

    @pl.when(i == _NGB - 1)
    def _fini():
        acc = jnp.float32(0.0)
        for k in range(_NRB):
            acc += part_ref[k, 0, 0]
        out_ref[0, 0] += acc


def kernel(x, target):
    tgt2 = target.reshape(_N, 1)
    s1 = pl.pallas_call(
        _stream_kernel,
        grid=(_NRB,),
        in_specs=[
            pl.BlockSpec((_RB, _V), lambda k: (k, 0)),
            pl.BlockSpec((_RB, 1), lambda k: (k, 0)),
        ],
        out_specs=pl.BlockSpec((1, 1, 1), lambda k: (k, 0, 0),
                               memory_space=pltpu.SMEM),
        out_shape=jax.ShapeDtypeStruct((_NRB, 1, 1), jnp.float32),
        compiler_params=pltpu.CompilerParams(
            dimension_semantics=("parallel",)),
    )(x, tgt2)

    out = pl.pallas_call(
        _gather_kernel,
        grid_spec=pltpu.PrefetchScalarGridSpec(
            num_scalar_prefetch=1,
            grid=(_NGB,),
            in_specs=[_make_gather_spec(j) for j in range(_GR)]
            + [pl.BlockSpec((_NRB, 1, 1), lambda i, tgt: (0, 0, 0),
                            memory_space=pltpu.SMEM)],
            out_specs=pl.BlockSpec((1, 1), lambda i, tgt: (0, 0),
                                   memory_space=pltpu.SMEM),
        ),
        out_shape=jax.ShapeDtypeStruct((1, 1), jnp.float32),
    )(target, *([x] * _GR), s1)

    return out[0, 0]


# speedup vs baseline: 1.0994x; 1.0994x over previous
import math

import jax
import jax.numpy as jnp
from jax import lax
from jax.experimental import pallas as pl
from jax.experimental.pallas import tpu as pltpu
from jax.experimental.pallas import tpu_sc as plsc

_V = 100000
_N = 1024
_PAD = 0
_SMOOTH = 0.1
_CONF = 1.0 - _SMOOTH
_FILL = _SMOOTH / (_V - 2)
_C = _FILL * math.log(_FILL) * (_V - 2) + _CONF * math.log(_CONF)

_NW = 32
_SC_ROWS = 512
_TC_ROWS = _N - _SC_ROWS
_RPW = _SC_ROWS // _NW

_CH = 13 * 128
_NCH = 60
_TAIL0 = _NCH * _CH
_TAILW = 128
_SC_COLS = _TAIL0 + _TAILW
_REM = _V - _SC_COLS

_RB = 32
_NRB = _TC_ROWS // _RB


def _stream_kernel(x_ref, tgt_ref, out_ref):
    xb = x_ref[...]
    rowsum = jnp.sum(xb, axis=1, keepdims=True)
    x0 = xb[:, 0:1]
    tgt = tgt_ref[...]
    cols = lax.broadcasted_iota(jnp.int32, xb.shape, 1)
    xt = jnp.sum(jnp.where(cols == tgt, xb, 0.0), axis=1, keepdims=True)
    valid = tgt != _PAD
    out_ref[0, 0, 0] = jnp.sum(
        jnp.where(valid,
                  _C - _FILL * (rowsum - x0) + (_FILL - _CONF) * xt,
                  0.0))


def _sc_kernel(x_hbm, tgt_hbm, rows_out, gath_out,
               buf0, buf1, tbuf, tgt_v, res_v, gbuf, gres_v,
               sem0, sem1, tsem, gsem):
    wid = lax.axis_index("s") * 2 + lax.axis_index("c")
    lanes = lax.iota(jnp.int32, 16)
    m0 = (1 - jnp.minimum(lanes, 1)).astype(jnp.float32)
    base = _TC_ROWS + wid * _RPW

    pltpu.sync_copy(tgt_hbm.at[pl.ds(base, _RPW)], tgt_v)
    tv = tgt_v[...]

    tvals = []
    c0s = []
    for r in range(_RPW):
        t = tv[r]
        c0 = jnp.minimum((t // 128) * 128, _TAIL0)
        tvals.append(t)
        c0s.append(c0)
        pltpu.async_copy(
            x_hbm.at[pl.ds(base + (r // 8) * 8, 8), pl.ds(c0, 128)],
            gbuf.at[r], gsem)

    tail_cp = pltpu.async_copy(
        x_hbm.at[pl.ds(base, _RPW), pl.ds(_TAIL0, _TAILW)], tbuf, tsem)
    pltpu.async_copy(x_hbm.at[pl.ds(base, _RPW), pl.ds(0, _CH)], buf0, sem0)
    accs = tuple(jnp.zeros((16,), jnp.float32) for _ in range(_RPW))
    for c in range(_NCH):
        cur, cur_sem = (buf0, sem0) if c % 2 == 0 else (buf1, sem1)
        nxt, nxt_sem = (buf1, sem1) if c % 2 == 0 else (buf0, sem0)
        if c + 1 < _NCH:
            pltpu.async_copy(
                x_hbm.at[pl.ds(base, _RPW), pl.ds((c + 1) * _CH, _CH)],
                nxt, nxt_sem)
        pltpu.make_async_copy(
            x_hbm.at[pl.ds(base, _RPW), pl.ds(c * _CH, _CH)],
            cur, cur_sem).wait()

        def body(i, a, _cur=cur):
            return tuple(a[r] + _cur[r, pl.ds(i * 16, 16)]
                         for r in range(_RPW))
        accs = lax.fori_loop(0, _CH // 16, body, accs)
        if c == 0:
            accs = tuple(
                accs[r] - cur[r, pl.ds(0, 16)] * m0
                for r in range(_RPW))

    tail_cp.wait()
    for i in range(_TAILW // 16):
        accs = tuple(accs[r] + tbuf[r, pl.ds(i * 16, 16)]
                     for r in range(_RPW))
    for r in range(_RPW):
        res_v[r] = accs[r]
    pltpu.sync_copy(res_v, rows_out.at[pl.ds(wid * _RPW, _RPW)])

    for r in range(_RPW):
        pltpu.make_async_copy(
            x_hbm.at[pl.ds(base + (r // 8) * 8, 8), pl.ds(c0s[r], 128)],
            gbuf.at[r], gsem).wait()
    acc_g = jnp.zeros((16,), jnp.float32)
    for r in range(_RPW):
        t = tvals[r]
        d = ((t % 128) // 16) * 16
        v = gbuf[r, r % 8, pl.ds(d, 16)]
        eq = (1 - jnp.minimum(jnp.abs(lanes - t % 16), 1)).astype(jnp.float32)
        w = jnp.where((t != _PAD) & (t < _SC_COLS), 1.0, 0.0)
        acc_g = acc_g + v * (eq * w)
    gres_v[...] = acc_g
    pltpu.sync_copy(gres_v, gath_out.at[wid])


def _combine_kernel(part_ref, rows_ref, gath_ref, tgt_ref, xtail_ref,
                    out_ref):
    xtail = xtail_ref[...]
    srow = (jnp.sum(rows_ref[...], axis=1, keepdims=True)
            + jnp.sum(xtail, axis=1, keepdims=True))
    tgt = tgt_ref[...]
    valid = tgt != _PAD
    s_sc = jnp.sum(jnp.where(valid, _C - _FILL * srow, 0.0))
    cols = lax.broadcasted_iota(jnp.int32, xtail.shape, 1) + _SC_COLS
    xt_tail = jnp.sum(jnp.where(tgt == cols, xtail, 0.0))
    s_g = (_FILL - _CONF) * (jnp.sum(gath_ref[...]) + xt_tail)
    s_tc = jnp.float32(0.0)
    for k in range(_NRB):
        s_tc += part_ref[k, 0, 0]
    out_ref[0, 0] = s_tc + s_sc + s_g


def kernel(x, target):
    tgt2 = target.reshape(_N, 1)

    s1 = pl.pallas_call(
        _stream_kernel,
        grid=(_NRB,),
        in_specs=[
            pl.BlockSpec((_RB, _V), lambda k: (k, 0)),
            pl.BlockSpec((_RB, 1), lambda k: (k, 0)),
        ],
        out_specs=pl.BlockSpec((1, 1, 1), lambda k: (k, 0, 0),
                               memory_space=pltpu.SMEM),
        out_shape=jax.ShapeDtypeStruct((_NRB, 1, 1), jnp.float32),
        compiler_params=pltpu.CompilerParams(
            dimension_semantics=("arbitrary",)),
    )(x, tgt2)

    sc_rows, sc_gath = pl.kernel(
        _sc_kernel,
        out_type=(
            jax.ShapeDtypeStruct((_SC_ROWS, 16), jnp.float32),
            jax.ShapeDtypeStruct((_NW, 16), jnp.float32),
        ),
        mesh=plsc.VectorSubcoreMesh(core_axis_name="c", subcore_axis_name="s"),
        scratch_types=[
            pltpu.VMEM((_RPW, _CH), jnp.float32),
            pltpu.VMEM((_RPW, _CH), jnp.float32),
            pltpu.VMEM((_RPW, _TAILW), jnp.float32),
            pltpu.VMEM((16,), jnp.int32),
            pltpu.VMEM((_RPW, 16), jnp.float32),
            pltpu.VMEM((_RPW, 8, 128), jnp.float32),
            pltpu.VMEM((16,), jnp.float32),
            pltpu.SemaphoreType.DMA,
            pltpu.SemaphoreType.DMA,
            pltpu.SemaphoreType.DMA,
            pltpu.SemaphoreType.DMA,
        ],
    )(x, target)

    xtail = lax.slice(x, (_TC_ROWS, _SC_COLS), (_N, _V))

    out = pl.pallas_call(
        _combine_kernel,
        in_specs=[
            pl.BlockSpec(memory_space=pltpu.SMEM),
            pl.BlockSpec(memory_space=pltpu.VMEM),
            pl.BlockSpec(memory_space=pltpu.VMEM),
            pl.BlockSpec(memory_space=pltpu.VMEM),
            pl.BlockSpec(memory_space=pltpu.VMEM),
        ],
        out_specs=pl.BlockSpec(memory_space=pltpu.SMEM),
        out_shape=jax.ShapeDtypeStruct((1, 1), jnp.float32),
    )(s1, sc_rows, sc_gath, tgt2[_TC_ROWS:], xtail)

    return out[0, 0]


# speedup vs baseline: 1.1010x; 1.0015x over previous
import math

import jax
import jax.numpy as jnp
from jax import lax
from jax.experimental import pallas as pl
from jax.experimental.pallas import tpu as pltpu
from jax.experimental.pallas import tpu_sc as plsc

_V = 100000
_N = 1024
_PAD = 0
_SMOOTH = 0.1
_CONF = 1.0 - _SMOOTH
_FILL = _SMOOTH / (_V - 2)
_C = _FILL * math.log(_FILL) * (_V - 2) + _CONF * math.log(_CONF)

_NW = 32
_SC_ROWS = 512
_TC_ROWS = _N - _SC_ROWS
_RPW = _SC_ROWS // _NW

_CH = 13 * 128
_NCH = 60
_TAIL0 = _NCH * _CH
_TAILW = 128
_SC_COLS = _TAIL0 + _TAILW
_REM = _V - _SC_COLS

_RB = 32
_NRB = _TC_ROWS // _RB


def _stream_kernel(x_ref, tgt_ref, out_ref):
    xb = x_ref[...]
    rowsum = jnp.sum(xb, axis=1, keepdims=True)
    x0 = xb[:, 0:1]
    tgt = tgt_ref[...]
    cols = lax.broadcasted_iota(jnp.int32, xb.shape, 1)
    xt = jnp.sum(jnp.where(cols == tgt, xb, 0.0), axis=1, keepdims=True)
    valid = tgt != _PAD
    out_ref[0, 0, 0] = jnp.sum(
        jnp.where(valid,
                  _C - _FILL * (rowsum - x0) + (_FILL - _CONF) * xt,
                  0.0))


def _sc_kernel(x_hbm, tgt_hbm, rows_out, gath_out,
               buf0, buf1, tbuf, tgt_v, res_v, gbuf, gres_v,
               sem0, sem1, tsem, gsem):
    wid = lax.axis_index("s") * 2 + lax.axis_index("c")
    lanes = lax.iota(jnp.int32, 16)
    m0 = (1 - jnp.minimum(lanes, 1)).astype(jnp.float32)
    base = _TC_ROWS + wid * _RPW

    pltpu.sync_copy(tgt_hbm.at[pl.ds(base, _RPW)], tgt_v)
    tv = tgt_v[...]

    tvals = []
    c0s = []
    for r in range(_RPW):
        t = tv[r]
        c0 = jnp.minimum((t // 128) * 128, _TAIL0)
        tvals.append(t)
        c0s.append(c0)
        pltpu.async_copy(
            x_hbm.at[pl.ds(base + (r // 8) * 8, 8), pl.ds(c0, 128)],
            gbuf.at[r], gsem)

    tail_cp = pltpu.async_copy(
        x_hbm.at[pl.ds(base, _RPW), pl.ds(_TAIL0, _TAILW)], tbuf, tsem)
    pltpu.async_copy(x_hbm.at[pl.ds(base, _RPW), pl.ds(0, _CH)], buf0, sem0)
    accs = tuple(jnp.zeros((16,), jnp.float32) for _ in range(_RPW))
    for c in range(_NCH):
        cur, cur_sem = (buf0, sem0) if c % 2 == 0 else (buf1, sem1)
        nxt, nxt_sem = (buf1, sem1) if c % 2 == 0 else (buf0, sem0)
        if c + 1 < _NCH:
            pltpu.async_copy(
                x_hbm.at[pl.ds(base, _RPW), pl.ds((c + 1) * _CH, _CH)],
                nxt, nxt_sem)
        pltpu.make_async_copy(
            x_hbm.at[pl.ds(base, _RPW), pl.ds(c * _CH, _CH)],
            cur, cur_sem).wait()

        def body(i, a, _cur=cur):
            return tuple(a[r] + _cur[r, pl.ds(i * 16, 16)]
                         for r in range(_RPW))
        accs = lax.fori_loop(0, _CH // 16, body, accs)
        if c == 0:
            accs = tuple(
                accs[r] - cur[r, pl.ds(0, 16)] * m0
                for r in range(_RPW))

    tail_cp.wait()
    for i in range(_TAILW // 16):
        accs = tuple(accs[r] + tbuf[r, pl.ds(i * 16, 16)]
                     for r in range(_RPW))
    for r in range(_RPW):
        res_v[r] = accs[r]
    pltpu.sync_copy(res_v, rows_out.at[pl.ds(wid * _RPW, _RPW)])

    for r in range(_RPW):
        pltpu.make_async_copy(
            x_hbm.at[pl.ds(base + (r // 8) * 8, 8), pl.ds(c0s[r], 128)],
            gbuf.at[r], gsem).wait()
    acc_g = jnp.zeros((16,), jnp.float32)
    for r in range(_RPW):
        t = tvals[r]
        d = ((t % 128) // 16) * 16
        v = gbuf[r, r % 8, pl.ds(d, 16)]
        eq = (1 - jnp.minimum(jnp.abs(lanes - t % 16), 1)).astype(jnp.float32)
        w = jnp.where((t != _PAD) & (t < _SC_COLS), 1.0, 0.0)
        acc_g = acc_g + v * (eq * w)
    gres_v[...] = acc_g
    pltpu.sync_copy(gres_v, gath_out.at[wid])


def _combine_kernel(part_ref, rows_ref, gath_ref, tgt_ref, xtail_ref,
                    out_ref):
    xtail = xtail_ref[...]
    srow = (jnp.sum(rows_ref[...], axis=1, keepdims=True)
            + jnp.sum(xtail, axis=1, keepdims=True))
    tgt = tgt_ref[...]
    valid = tgt != _PAD
    s_sc = jnp.sum(jnp.where(valid, _C - _FILL * srow, 0.0))
    cols = lax.broadcasted_iota(jnp.int32, xtail.shape, 1) + _SC_COLS
    xt_tail = jnp.sum(jnp.where(tgt == cols, xtail, 0.0))
    s_g = (_FILL - _CONF) * (jnp.sum(gath_ref[...]) + xt_tail)
    s_tc = jnp.float32(0.0)
    for k in range(_NRB):
        s_tc += part_ref[k, 0, 0]
    out_ref[0, 0] = s_tc + s_sc + s_g


def kernel(x, target):
    tgt2 = target.reshape(_N, 1)

    sc_rows, sc_gath = pl.kernel(
        _sc_kernel,
        out_type=(
            jax.ShapeDtypeStruct((_SC_ROWS, 16), jnp.float32),
            jax.ShapeDtypeStruct((_NW, 16), jnp.float32),
        ),
        mesh=plsc.VectorSubcoreMesh(core_axis_name="c", subcore_axis_name="s"),
        scratch_types=[
            pltpu.VMEM((_RPW, _CH), jnp.float32),
            pltpu.VMEM((_RPW, _CH), jnp.float32),
            pltpu.VMEM((_RPW, _TAILW), jnp.float32),
            pltpu.VMEM((16,), jnp.int32),
            pltpu.VMEM((_RPW, 16), jnp.float32),
            pltpu.VMEM((_RPW, 8, 128), jnp.float32),
            pltpu.VMEM((16,), jnp.float32),
            pltpu.SemaphoreType.DMA,
            pltpu.SemaphoreType.DMA,
            pltpu.SemaphoreType.DMA,
            pltpu.SemaphoreType.DMA,
        ],
    )(x, target)

    s1 = pl.pallas_call(
        _stream_kernel,
        grid=(_NRB,),
        in_specs=[
            pl.BlockSpec((_RB, _V), lambda k: (k, 0)),
            pl.BlockSpec((_RB, 1), lambda k: (k, 0)),
        ],
        out_specs=pl.BlockSpec((1, 1, 1), lambda k: (k, 0, 0),
                               memory_space=pltpu.SMEM),
        out_shape=jax.ShapeDtypeStruct((_NRB, 1, 1), jnp.float32),
        compiler_params=pltpu.CompilerParams(
            dimension_semantics=("arbitrary",)),
    )(x, tgt2)

    xtail = lax.slice(x, (_TC_ROWS, _SC_COLS), (_N, _V))

    out = pl.pallas_call(
        _combine_kernel,
        in_specs=[
            pl.BlockSpec(memory_space=pltpu.SMEM),
            pl.BlockSpec(memory_space=pltpu.VMEM),
            pl.BlockSpec(memory_space=pltpu.VMEM),
            pl.BlockSpec(memory_space=pltpu.VMEM),
            pl.BlockSpec(memory_space=pltpu.VMEM),
        ],
        out_specs=pl.BlockSpec(memory_space=pltpu.SMEM),
        out_shape=jax.ShapeDtypeStruct((1, 1), jnp.float32),
    )(s1, sc_rows, sc_gath, tgt2[_TC_ROWS:], xtail)

    return out[0, 0]
